# baseline (speedup 1.0000x reference)
import jax
import jax.numpy as jnp
from jax.experimental import pallas as pl


def _h_kernel(x_ref, w_ref, b_ref, h_ref):
    h = jnp.dot(x_ref[...], w_ref[...], preferred_element_type=jnp.float32)
    h_ref[...] = (h + b_ref[...]).astype(jnp.bfloat16)


def _adj_mm_kernel(adj_ref, h_ref, o_ref):
    a = adj_ref[...].astype(jnp.bfloat16)
    o_ref[...] = jnp.dot(a, h_ref[...], preferred_element_type=jnp.float32)


def kernel(input, adj, weight, bias):
    n, _ = input.shape
    d_out = weight.shape[1]
    m = adj.shape[0]

    h = pl.pallas_call(
        _h_kernel,
        out_shape=jax.ShapeDtypeStruct((n, d_out), jnp.bfloat16),
    )(input, weight, bias.reshape(1, d_out))

    tile_m = 400
    out = pl.pallas_call(
        _adj_mm_kernel,
        grid=(m // tile_m,),
        in_specs=[
            pl.BlockSpec((tile_m, n), lambda i: (i, 0)),
            pl.BlockSpec((n, d_out), lambda i: (0, 0)),
        ],
        out_specs=pl.BlockSpec((tile_m, d_out), lambda i: (i, 0)),
        out_shape=jax.ShapeDtypeStruct((m, d_out), jnp.float32),
    )(adj, h)
    return out


# speedup vs baseline: 1.0413x; 1.0413x over previous
import jax
import jax.numpy as jnp
from jax.experimental import pallas as pl
from jax.experimental.pallas import tpu as pltpu


def _fused_kernel(adj_ref, x_ref, w_ref, b_ref, o_ref, h_scr):
    @pl.when(pl.program_id(0) == 0)
    def _():
        h = jnp.dot(x_ref[...], w_ref[...], preferred_element_type=jnp.float32)
        h_scr[...] = (h + b_ref[...]).astype(jnp.bfloat16)

    a = adj_ref[...].astype(jnp.bfloat16)
    o_ref[...] = jnp.dot(a, h_scr[...], preferred_element_type=jnp.float32)


def kernel(input, adj, weight, bias):
    n, d_in = input.shape
    d_out = weight.shape[1]
    m = adj.shape[0]

    tile_m = 400
    out = pl.pallas_call(
        _fused_kernel,
        grid=(m // tile_m,),
        in_specs=[
            pl.BlockSpec((tile_m, n), lambda i: (i, 0)),
            pl.BlockSpec((n, d_in), lambda i: (0, 0)),
            pl.BlockSpec((d_in, d_out), lambda i: (0, 0)),
            pl.BlockSpec((1, d_out), lambda i: (0, 0)),
        ],
        out_specs=pl.BlockSpec((tile_m, d_out), lambda i: (i, 0)),
        out_shape=jax.ShapeDtypeStruct((m, d_out), jnp.float32),
        scratch_shapes=[pltpu.VMEM((n, d_out), jnp.bfloat16)],
    )(adj, input, weight, bias.reshape(1, d_out))
    return out
